# baseline (speedup 1.0000x reference)
import functools
import math

import jax
import jax.numpy as jnp
from jax import lax
from jax.experimental import pallas as pl
from jax.experimental.pallas import tpu as pltpu
from jax.experimental.pallas import tpu_sc as plsc

D_MODEL = 64
SCALE = math.sqrt(D_MODEL)

NUM_CORES = 2
NUM_SUBCORES = 16
NUM_WORKERS = NUM_CORES * NUM_SUBCORES
LANES = 16

CHUNK = 128
NBUF = 2


def _emb_body(idx_hbm, tbl_hbm, out_hbm, idxall, pairbuf, obuf, gsem, osem):
    per_w = idxall.shape[0]
    wid = lax.axis_index("s") * NUM_CORES + lax.axis_index("c")
    g0 = wid * per_w
    pltpu.sync_copy(idx_hbm.at[pl.ds(g0, per_w)], idxall)

    iota16 = lax.iota(jnp.int32, 16)

    def gather(j, b):
        return pltpu.make_async_copy(
            tbl_hbm.at[idxall.at[j]],
            pairbuf.at[b, :, pl.ds(0, 2 * D_MODEL)], gsem.at[b])

    def writeback(j, b):
        g = g0 + j
        s = lax.shift_right_logical(g, 7)
        b0 = pl.multiple_of(lax.shift_left(lax.bitwise_and(g, 127), 7), CHUNK)
        return pltpu.make_async_copy(
            obuf.at[b], out_hbm.at[s, :, pl.ds(b0, CHUNK)], osem.at[b])

    def transpose_scale(b):
        def tblock(jb, carry):
            base = jb * LANES
            sl = pl.ds(base, LANES)
            rowv = iota16 + base

            @plsc.parallel_loop(0, D_MODEL, 1, unroll=16)
            def _d(d):
                colv = lax.broadcast(d, (LANES,))
                v = plsc.load_gather(pairbuf.at[b], [rowv, colv])
                obuf[b, d, sl] = v

            return carry

        lax.fori_loop(0, CHUNK // LANES, tblock, 0)

    for b in range(NBUF):
        gather(b, b).start()

    n_groups = per_w // NBUF

    def group(it, carry):
        for b in range(NBUF):
            j = it * NBUF + b
            gather(j, b).wait()

            @pl.when(it > 0)
            def _wait_wb():
                writeback(j, b).wait()

            transpose_scale(b)
            writeback(j, b).start()

            @pl.when(j < per_w - NBUF)
            def _refill():
                gather(j + NBUF, b).start()

        return carry

    lax.fori_loop(0, n_groups, group, 0)
    for b in range(NBUF):
        writeback(per_w - NBUF + b, b).wait()


def _relayout_body(src_ref, dst_ref):
    dst_ref[:, 0:D_MODEL] = jnp.transpose(src_ref[...], (1, 0)) * SCALE


def _prep_table(lut_t):
    v = lut_t.shape[1]
    return pl.pallas_call(
        _relayout_body,
        out_shape=jax.ShapeDtypeStruct((v, 2 * D_MODEL), jnp.float32),
        grid=(pl.cdiv(v, 2 * D_MODEL),),
        in_specs=[pl.BlockSpec((D_MODEL, 2 * D_MODEL), lambda i: (0, i))],
        out_specs=pl.BlockSpec((2 * D_MODEL, 2 * D_MODEL), lambda i: (i, 0)),
    )(lut_t)


def kernel(x, lut_weight):
    b0, b1 = x.shape
    total = b0 * b1
    n_chunks = total // CHUNK
    per_w = n_chunks // NUM_WORKERS
    idx2d = jnp.transpose(x).reshape(n_chunks, CHUNK).astype(jnp.int32)
    tblp = _prep_table(jnp.transpose(lut_weight))

    mesh = plsc.VectorSubcoreMesh(core_axis_name="c", subcore_axis_name="s")
    emb = functools.partial(
        pl.kernel,
        mesh=mesh,
        out_type=jax.ShapeDtypeStruct((b1, D_MODEL, b0), jnp.float32),
        scratch_types=[
            pltpu.VMEM((per_w, CHUNK), jnp.int32),
            pltpu.VMEM((NBUF, CHUNK, 2 * D_MODEL + 1), jnp.float32),
            pltpu.VMEM((NBUF, D_MODEL, CHUNK), jnp.float32),
            pltpu.SemaphoreType.DMA((NBUF,)),
            pltpu.SemaphoreType.DMA((NBUF,)),
        ],
        compiler_params=pltpu.CompilerParams(needs_layout_passes=False),
    )(_emb_body)
    res = emb(idx2d, tblp)
    return jnp.transpose(res, (2, 0, 1))


# speedup vs baseline: 3.9078x; 3.9078x over previous
import functools
import math

import jax
import jax.numpy as jnp
from jax import lax
from jax.experimental import pallas as pl
from jax.experimental.pallas import tpu as pltpu
from jax.experimental.pallas import tpu_sc as plsc

D_MODEL = 64
SCALE = math.sqrt(D_MODEL)

NUM_CORES = 2
NUM_SUBCORES = 16
NUM_WORKERS = NUM_CORES * NUM_SUBCORES
LANES = 16

CHUNK = 128
NBUF = 2


def _emb_body(idx_hbm, tbl_hbm, out_hbm, idxall, pairbuf, obuf, gsem, osem):
    per_w = idxall.shape[0]
    wid = lax.axis_index("s") * NUM_CORES + lax.axis_index("c")
    g0 = wid * per_w
    pltpu.sync_copy(idx_hbm.at[pl.ds(g0, per_w)], idxall)

    iota16 = lax.iota(jnp.int32, 16)

    def gather(j, b):
        return pltpu.make_async_copy(
            tbl_hbm.at[idxall.at[j]],
            pairbuf.at[b, :, pl.ds(0, 2 * D_MODEL)], gsem.at[b])

    def writeback(j, b):
        g = g0 + j
        s = lax.shift_right_logical(g, 7)
        b0 = pl.multiple_of(lax.shift_left(lax.bitwise_and(g, 127), 7), CHUNK)
        return pltpu.make_async_copy(
            obuf.at[b], out_hbm.at[s, :, pl.ds(b0, CHUNK)], osem.at[b])

    def transpose_scale(b):
        def tblock(jb, carry):
            base = jb * LANES
            sl = pl.ds(base, LANES)
            rowv = iota16 + base

            @plsc.parallel_loop(0, D_MODEL, 1, unroll=16)
            def _d(d):
                colv = lax.broadcast(d, (LANES,))
                v = plsc.load_gather(pairbuf.at[b], [rowv, colv])
                obuf[b, d, sl] = v * SCALE

            return carry

        lax.fori_loop(0, CHUNK // LANES, tblock, 0)

    for b in range(NBUF):
        gather(b, b).start()

    n_groups = per_w // NBUF

    def group(it, carry):
        for b in range(NBUF):
            j = it * NBUF + b
            gather(j, b).wait()

            @pl.when(it > 0)
            def _wait_wb():
                writeback(j, b).wait()

            transpose_scale(b)
            writeback(j, b).start()

            @pl.when(j < per_w - NBUF)
            def _refill():
                gather(j + NBUF, b).start()

        return carry

    lax.fori_loop(0, n_groups, group, 0)
    for b in range(NBUF):
        writeback(per_w - NBUF + b, b).wait()


def kernel(x, lut_weight):
    b0, b1 = x.shape
    total = b0 * b1
    n_chunks = total // CHUNK
    per_w = n_chunks // NUM_WORKERS
    idx2d = jnp.transpose(x).reshape(n_chunks, CHUNK).astype(jnp.int32)
    tblp = jnp.pad(lut_weight, ((0, 0), (0, 2 * D_MODEL - lut_weight.shape[1])))

    mesh = plsc.VectorSubcoreMesh(core_axis_name="c", subcore_axis_name="s")
    emb = functools.partial(
        pl.kernel,
        mesh=mesh,
        out_type=jax.ShapeDtypeStruct((b1, D_MODEL, b0), jnp.float32),
        scratch_types=[
            pltpu.VMEM((per_w, CHUNK), jnp.int32),
            pltpu.VMEM((NBUF, CHUNK, 2 * D_MODEL + 1), jnp.float32),
            pltpu.VMEM((NBUF, D_MODEL, CHUNK), jnp.float32),
            pltpu.SemaphoreType.DMA((NBUF,)),
            pltpu.SemaphoreType.DMA((NBUF,)),
        ],
        compiler_params=pltpu.CompilerParams(needs_layout_passes=False),
    )(_emb_body)
    res = emb(idx2d, tblp)
    return jnp.transpose(res, (2, 0, 1))
